# baseline (speedup 1.0000x reference)
import functools

import jax
import jax.numpy as jnp
from jax.experimental import pallas as pl
from jax.experimental.pallas import tpu as pltpu

_BI = 400


def _sweep1_body(x_ref, adj_ref, w1_ref, b1_ref, wc_ref,
                 s2_ref, adjh_ref, sup_ref, *, nb, bi):
    t = pl.program_id(0)

    @pl.when(t == 0)
    def _():
        sup_ref[...] = jnp.dot(x_ref[...], w1_ref[...],
                               preferred_element_type=jnp.float32)

    @pl.when(t >= 1)
    def _():
        a = adj_ref[...]
        adjh_ref[...] = a.astype(jnp.bfloat16)
        h = jnp.dot(a, sup_ref[...], preferred_element_type=jnp.float32)
        h = jnp.maximum(h + b1_ref[...], 0.0)
        s2_ref[...] = jnp.dot(h, wc_ref[...],
                              preferred_element_type=jnp.float32)


def _sweep2_body(adjh_ref, s2_ref, bc_ref, eps_ref, out_ref, *, nclass):
    acc = jnp.dot(adjh_ref[...], s2_ref[...],
                  preferred_element_type=jnp.float32)
    acc = acc + bc_ref[...]
    mean = acc[:, :nclass]
    logstd = acc[:, nclass:]
    z = eps_ref[...] * jnp.exp(logstd) + mean
    m = jnp.max(z, axis=1, keepdims=True)
    zs = z - m
    lse = jnp.log(jnp.sum(jnp.exp(zs), axis=1, keepdims=True))
    out_ref[...] = zs - lse


def kernel(x, adj, W1, b1, W11, b11, W12, b12):
    n, nfeat = x.shape
    nhid = W1.shape[1]
    nclass = W11.shape[1]
    nc2 = 2 * nclass

    bi = _BI if n % _BI == 0 else 8
    nb = n // bi

    wc = jnp.concatenate([W11, W12], axis=1)
    bc = jnp.concatenate([b11, b12])[None, :]
    b1r = b1[None, :]
    eps = jax.random.normal(jax.random.key(42), (n, nclass), dtype=jnp.float32)

    def shift_map(t):
        return (jnp.where(t == 0, 0, t - 1), 0)

    s2, adjh = pl.pallas_call(
        functools.partial(_sweep1_body, nb=nb, bi=bi),
        grid=(nb + 1,),
        in_specs=[
            pl.BlockSpec((n, nfeat), lambda t: (0, 0)),
            pl.BlockSpec((bi, n), shift_map),
            pl.BlockSpec((nfeat, nhid), lambda t: (0, 0)),
            pl.BlockSpec((1, nhid), lambda t: (0, 0)),
            pl.BlockSpec((nhid, nc2), lambda t: (0, 0)),
        ],
        out_specs=[
            pl.BlockSpec((bi, nc2), shift_map),
            pl.BlockSpec((bi, n), shift_map),
        ],
        out_shape=[
            jax.ShapeDtypeStruct((n, nc2), jnp.float32),
            jax.ShapeDtypeStruct((n, n), jnp.bfloat16),
        ],
        scratch_shapes=[
            pltpu.VMEM((n, nhid), jnp.float32),
        ],
    )(x, adj, W1, b1r, wc)

    out = pl.pallas_call(
        functools.partial(_sweep2_body, nclass=nclass),
        grid=(nb,),
        in_specs=[
            pl.BlockSpec((bi, n), lambda i: (i, 0)),
            pl.BlockSpec((n, nc2), lambda i: (0, 0)),
            pl.BlockSpec((1, nc2), lambda i: (0, 0)),
            pl.BlockSpec((bi, nclass), lambda i: (i, 0)),
        ],
        out_specs=pl.BlockSpec((bi, nclass), lambda i: (i, 0)),
        out_shape=jax.ShapeDtypeStruct((n, nclass), jnp.float32),
    )(adjh, s2.astype(jnp.bfloat16), bc, eps)

    return out


# speedup vs baseline: 1.0924x; 1.0924x over previous
import functools

import jax
import jax.numpy as jnp
from jax.experimental import pallas as pl
from jax.experimental.pallas import tpu as pltpu

_BI = 400


def _body(x_ref, adj_ref, w1_ref, b1_ref, wc_ref, bc_ref, eps_ref,
          out_ref, sup_ref, s2_ref, *, nb, bi, nclass):
    t = pl.program_id(0)

    @pl.when(t == 0)
    def _():
        sup_ref[...] = jnp.dot(x_ref[...], w1_ref[...],
                               preferred_element_type=jnp.float32)

    @pl.when((t >= 1) & (t <= nb))
    def _():
        h = jnp.dot(adj_ref[...], sup_ref[...],
                    preferred_element_type=jnp.float32)
        h = jnp.maximum(h + b1_ref[...], 0.0)
        s2_ref[pl.ds((t - 1) * bi, bi), :] = jnp.dot(
            h, wc_ref[...], preferred_element_type=jnp.float32)

    @pl.when(t > nb)
    def _():
        acc = jnp.dot(adj_ref[...], s2_ref[...],
                      preferred_element_type=jnp.float32)
        acc = acc + bc_ref[...]
        mean = acc[:, :nclass]
        logstd = acc[:, nclass:]
        z = eps_ref[...] * jnp.exp(logstd) + mean
        m = jnp.max(z, axis=1, keepdims=True)
        zs = z - m
        lse = jnp.log(jnp.sum(jnp.exp(zs), axis=1, keepdims=True))
        out_ref[...] = zs - lse


def kernel(x, adj, W1, b1, W11, b11, W12, b12):
    n, nfeat = x.shape
    nhid = W1.shape[1]
    nclass = W11.shape[1]
    nc2 = 2 * nclass

    bi = _BI if n % _BI == 0 else 8
    nb = n // bi

    wc = jnp.concatenate([W11, W12], axis=1)
    bc = jnp.concatenate([b11, b12])[None, :]
    b1r = b1[None, :]
    eps = jax.random.normal(jax.random.key(42), (n, nclass), dtype=jnp.float32)

    def adj_map(t):
        return (jnp.where(t == 0, 0, jnp.where(t <= nb, t - 1, t - 1 - nb)), 0)

    def row_map(t):
        return (jnp.where(t > nb, t - 1 - nb, 0), 0)

    out = pl.pallas_call(
        functools.partial(_body, nb=nb, bi=bi, nclass=nclass),
        grid=(2 * nb + 1,),
        in_specs=[
            pl.BlockSpec((n, nfeat), lambda t: (0, 0)),
            pl.BlockSpec((bi, n), adj_map),
            pl.BlockSpec((nfeat, nhid), lambda t: (0, 0)),
            pl.BlockSpec((1, nhid), lambda t: (0, 0)),
            pl.BlockSpec((nhid, nc2), lambda t: (0, 0)),
            pl.BlockSpec((1, nc2), lambda t: (0, 0)),
            pl.BlockSpec((bi, nclass), row_map),
        ],
        out_specs=pl.BlockSpec((bi, nclass), row_map),
        out_shape=jax.ShapeDtypeStruct((n, nclass), jnp.float32),
        scratch_shapes=[
            pltpu.VMEM((n, nhid), jnp.float32),
            pltpu.VMEM((n, nc2), jnp.float32),
        ],
    )(x, adj, W1, b1r, wc, bc, eps)

    return out
